# Initial kernel scaffold; baseline (speedup 1.0000x reference)
import jax
import jax.numpy as jnp
from jax.experimental import pallas as pl


def kernel(pred_sid, pred_energy_corr_factor, pred_beta, recHitEnergy, recHitID, alpha_idx_tracks, alpha_idx_hits):
    raise NotImplementedError("write your pallas kernel here")


# speedup vs baseline: 58.3402x; 58.3402x over previous
import functools

import jax
import jax.numpy as jnp
from jax import lax
from jax.experimental import pallas as pl
from jax.experimental.pallas import tpu as pltpu
from jax.experimental.pallas import tpu_sc as plsc

N = 200000
S = 5000
NS = 16
L = 16

CH = 12544
NP = NS * CH
SP = 5120
AP = 384
APT = NS * AP
ZB = SP // NS

_mesh = plsc.VectorSubcoreMesh(core_axis_name="c", subcore_axis_name="s")


@functools.partial(
    pl.kernel,
    mesh=_mesh,
    compiler_params=pltpu.CompilerParams(needs_layout_passes=False),
    out_type=[jax.ShapeDtypeStruct((2 * NP,), jnp.float32) for _ in range(2)],
    scratch_types=[
        pltpu.VMEM((CH,), jnp.int32),
        pltpu.VMEM((CH,), jnp.int32),
        pltpu.VMEM((CH,), jnp.float32),
        pltpu.VMEM((CH,), jnp.float32),
        pltpu.VMEM((CH,), jnp.float32),
        pltpu.VMEM((CH,), jnp.float32),
        pltpu.VMEM((SP,), jnp.float32),
        pltpu.VMEM((APT,), jnp.float32),
        pltpu.VMEM((AP,), jnp.int32),
        pltpu.VMEM((AP,), jnp.float32),
        pltpu.VMEM_SHARED((SP,), jnp.float32),
        pltpu.VMEM_SHARED((APT,), jnp.float32),
        pltpu.SemaphoreType.DMA,
    ],
)
def _sc_kernel(sid_h, id_h, e_h, pecf_h, alpha_h,
               raw_out_h, cor_out_h,
               sid_v, id_v, e_v, vals_v, raw_v, cor_v, sums_v, corr_v,
               aidx_v, acorr_v, sums_sh, corr_sh, sem):
    c = lax.axis_index("c")
    s = lax.axis_index("s")
    base = s * CH

    pltpu.sync_copy(sid_h.at[pl.ds(base, CH)], sid_v)
    pltpu.sync_copy(id_h.at[pl.ds(base, CH)], id_v)
    pltpu.sync_copy(e_h.at[pl.ds(base, CH)], e_v)

    def _zbody(j, carry):
        sums_v[pl.ds(j * L, L)] = jnp.zeros((L,), jnp.float32)
        return carry
    lax.fori_loop(0, ZB // L, _zbody, 0)
    pltpu.sync_copy(sums_v.at[pl.ds(0, ZB)], sums_sh.at[pl.ds(s * ZB, ZB)])

    pltpu.sync_copy(alpha_h.at[pl.ds(c * APT + s * AP, AP)], aidx_v)
    pltpu.async_copy(pecf_h.at[aidx_v], acorr_v, sem).wait()
    pltpu.sync_copy(acorr_v, corr_sh.at[pl.ds(s * AP, AP)])

    def _vbody(j, carry):
        id16 = id_v[pl.ds(j * L, L)]
        e16 = e_v[pl.ds(j * L, L)]
        vals_v[pl.ds(j * L, L)] = jnp.where(id16 == c, e16, jnp.zeros((L,), jnp.float32))
        return carry
    lax.fori_loop(0, CH // L, _vbody, 0)

    plsc.subcore_barrier()

    pltpu.sync_copy(vals_v, sums_sh.at[sid_v], add=True)

    plsc.subcore_barrier()

    pltpu.sync_copy(sums_sh, sums_v)
    pltpu.sync_copy(corr_sh, corr_v)

    def _gbody(j, carry):
        sid16 = sid_v[pl.ds(j * L, L)]
        raw = plsc.load_gather(sums_v, [sid16])
        cfac = plsc.load_gather(corr_v, [sid16])
        raw_v[pl.ds(j * L, L)] = raw
        cor_v[pl.ds(j * L, L)] = raw * cfac
        return carry
    lax.fori_loop(0, CH // L, _gbody, 0)

    pltpu.sync_copy(raw_v, raw_out_h.at[pl.ds(c * NP + base, CH)])
    pltpu.sync_copy(cor_v, cor_out_h.at[pl.ds(c * NP + base, CH)])


def kernel(pred_sid, pred_energy_corr_factor, pred_beta, recHitEnergy,
           recHitID, alpha_idx_tracks, alpha_idx_hits):
    del pred_beta
    sid = pred_sid[:, 0]
    hid = recHitID[:, 0]
    e = recHitEnergy[:, 0]
    pecf = pred_energy_corr_factor[:, 0]

    pad = NP - N
    sid_p = jnp.concatenate([sid, jnp.zeros((pad,), jnp.int32)])
    id_p = jnp.concatenate([hid, jnp.zeros((pad,), jnp.int32)])
    e_p = jnp.concatenate([e, jnp.zeros((pad,), jnp.float32)])
    apad = (jnp.arange(APT - S, dtype=jnp.int32) * 97) % N
    alpha_all = jnp.concatenate([
        alpha_idx_hits.astype(jnp.int32), apad,
        alpha_idx_tracks.astype(jnp.int32), apad,
    ])

    raw_all, cor_all = _sc_kernel(sid_p, id_p, e_p, pecf, alpha_all)
    out = lambda a, o: lax.dynamic_slice(a, (o,), (N,)).reshape(N, 1)
    return (out(raw_all, NP), out(cor_all, NP), out(raw_all, 0), out(cor_all, 0))


# speedup vs baseline: 64.3038x; 1.1022x over previous
import functools

import jax
import jax.numpy as jnp
from jax import lax
from jax.experimental import pallas as pl
from jax.experimental.pallas import tpu as pltpu
from jax.experimental.pallas import tpu_sc as plsc

N = 200000
S = 5000
NS = 16
L = 16

CH = 12544
OV = NS * CH - N
OV16 = OV // L
SP = 5120
AP = 320
ZB = SP // NS

_mesh = plsc.VectorSubcoreMesh(core_axis_name="c", subcore_axis_name="s")


@functools.partial(
    pl.kernel,
    mesh=_mesh,
    compiler_params=pltpu.CompilerParams(needs_layout_passes=False),
    out_type=[jax.ShapeDtypeStruct((N,), jnp.float32) for _ in range(4)],
    scratch_types=[
        pltpu.VMEM((CH,), jnp.int32),
        pltpu.VMEM((CH,), jnp.int32),
        pltpu.VMEM((CH,), jnp.float32),
        pltpu.VMEM((CH,), jnp.float32),
        pltpu.VMEM((CH,), jnp.float32),
        pltpu.VMEM((CH,), jnp.float32),
        pltpu.VMEM((SP,), jnp.float32),
        pltpu.VMEM((SP,), jnp.float32),
        pltpu.VMEM((AP,), jnp.int32),
        pltpu.VMEM((AP,), jnp.float32),
        pltpu.VMEM_SHARED((SP,), jnp.float32),
        pltpu.VMEM_SHARED((SP,), jnp.float32),
        pltpu.SemaphoreType.DMA,
    ],
)
def _sc_kernel(sid_h, id_h, e_h, pecf_h, alpha_h,
               tr_raw_h, tr_cor_h, hi_raw_h, hi_cor_h,
               sid_v, id_v, e_v, vals_v, raw_v, cor_v, sums_v, corr_v,
               aidx_v, acorr_v, sums_sh, corr_sh, sem):
    c = lax.axis_index("c")
    s = lax.axis_index("s")
    is_last = s == NS - 1
    base = jnp.where(is_last, N - CH, s * CH)
    abase = jnp.where(is_last, S - AP, s * AP)

    pltpu.sync_copy(alpha_h.at[pl.ds(c * S + abase, AP)], aidx_v)
    agather = pltpu.async_copy(pecf_h.at[aidx_v], acorr_v, sem)

    pltpu.sync_copy(sid_h.at[pl.ds(base, CH)], sid_v)
    pltpu.sync_copy(id_h.at[pl.ds(base, CH)], id_v)
    pltpu.sync_copy(e_h.at[pl.ds(base, CH)], e_v)

    def _zbody(j, carry):
        sums_v[pl.ds(j * L, L)] = jnp.zeros((L,), jnp.float32)
        return carry
    lax.fori_loop(0, ZB // L, _zbody, 0)
    pltpu.sync_copy(sums_v.at[pl.ds(0, ZB)], sums_sh.at[pl.ds(s * ZB, ZB)])

    agather.wait()
    pltpu.sync_copy(acorr_v, corr_sh.at[pl.ds(abase, AP)])

    jmin = jnp.where(is_last, OV16, 0)

    @plsc.parallel_loop(0, CH // L, unroll=8)
    def _vbody(j):
        id16 = id_v[pl.ds(j * L, L)]
        e16 = e_v[pl.ds(j * L, L)]
        keep = jnp.logical_and(id16 == c, j >= jmin)
        vals_v[pl.ds(j * L, L)] = jnp.where(keep, e16, jnp.zeros((L,), jnp.float32))

    plsc.subcore_barrier()

    pltpu.sync_copy(vals_v, sums_sh.at[sid_v], add=True)

    plsc.subcore_barrier()

    pltpu.sync_copy(sums_sh, sums_v)
    pltpu.sync_copy(corr_sh, corr_v)

    @plsc.parallel_loop(0, CH // L, unroll=8)
    def _gbody(j):
        sid16 = sid_v[pl.ds(j * L, L)]
        raw = plsc.load_gather(sums_v, [sid16])
        cfac = plsc.load_gather(corr_v, [sid16])
        raw_v[pl.ds(j * L, L)] = raw
        cor_v[pl.ds(j * L, L)] = raw * cfac

    @pl.when(c == 0)
    def _():
        pltpu.sync_copy(raw_v, hi_raw_h.at[pl.ds(base, CH)])
        pltpu.sync_copy(cor_v, hi_cor_h.at[pl.ds(base, CH)])

    @pl.when(c == 1)
    def _():
        pltpu.sync_copy(raw_v, tr_raw_h.at[pl.ds(base, CH)])
        pltpu.sync_copy(cor_v, tr_cor_h.at[pl.ds(base, CH)])


def kernel(pred_sid, pred_energy_corr_factor, pred_beta, recHitEnergy,
           recHitID, alpha_idx_tracks, alpha_idx_hits):
    del pred_beta
    sid = pred_sid[:, 0]
    hid = recHitID[:, 0]
    e = recHitEnergy[:, 0]
    pecf = pred_energy_corr_factor[:, 0]
    alpha_all = jnp.concatenate([alpha_idx_hits.astype(jnp.int32),
                                 alpha_idx_tracks.astype(jnp.int32)])

    tr_raw, tr_cor, hi_raw, hi_cor = _sc_kernel(sid, hid, e, pecf, alpha_all)
    out = lambda a: a.reshape(N, 1)
    return (out(tr_raw), out(tr_cor), out(hi_raw), out(hi_cor))


# speedup vs baseline: 64.3897x; 1.0013x over previous
import functools

import jax
import jax.numpy as jnp
from jax import lax
from jax.experimental import pallas as pl
from jax.experimental.pallas import tpu as pltpu
from jax.experimental.pallas import tpu_sc as plsc

N = 200000
S = 5000
NS = 16
L = 16

CH = 12544
OV = NS * CH - N
OV16 = OV // L
SP = 5120
AP = 320
ZB = SP // NS

_mesh = plsc.VectorSubcoreMesh(core_axis_name="c", subcore_axis_name="s")


@functools.partial(
    pl.kernel,
    mesh=_mesh,
    compiler_params=pltpu.CompilerParams(needs_layout_passes=False,
                                         use_tc_tiling_on_sc=False),
    out_type=[jax.ShapeDtypeStruct((N,), jnp.float32) for _ in range(4)],
    scratch_types=[
        pltpu.VMEM((CH,), jnp.int32),
        pltpu.VMEM((CH,), jnp.int32),
        pltpu.VMEM((CH,), jnp.float32),
        pltpu.VMEM((CH,), jnp.float32),
        pltpu.VMEM((CH,), jnp.float32),
        pltpu.VMEM((CH,), jnp.float32),
        pltpu.VMEM((SP,), jnp.float32),
        pltpu.VMEM((SP,), jnp.float32),
        pltpu.VMEM((AP,), jnp.int32),
        pltpu.VMEM((AP,), jnp.float32),
        pltpu.VMEM_SHARED((SP,), jnp.float32),
        pltpu.VMEM_SHARED((SP,), jnp.float32),
        pltpu.SemaphoreType.DMA,
    ],
)
def _sc_kernel(sid_h, id_h, e_h, pecf_h, ah_h,
               tr_raw_h, tr_cor_h, hi_raw_h, hi_cor_h,
               sid_v, id_v, e_v, vals_v, raw_v, cor_v, sums_v, corr_v,
               aidx_v, acorr_v, sums_sh, corr_sh, sem):
    c = lax.axis_index("c")
    s = lax.axis_index("s")
    is_last = s == NS - 1
    base = jnp.where(is_last, N - CH, s * CH)
    abase = jnp.where(is_last, S - AP, s * AP)

    pltpu.sync_copy(ah_h.at[pl.ds(c * S + abase, AP)], aidx_v)
    agather = pltpu.async_copy(pecf_h.at[aidx_v], acorr_v, sem)

    pltpu.sync_copy(sid_h.at[pl.ds(base, CH)], sid_v)
    pltpu.sync_copy(id_h.at[pl.ds(base, CH)], id_v)
    pltpu.sync_copy(e_h.at[pl.ds(base, CH)], e_v)

    def _zbody(j, carry):
        sums_v[pl.ds(j * L, L)] = jnp.zeros((L,), jnp.float32)
        return carry
    lax.fori_loop(0, ZB // L, _zbody, 0)
    pltpu.sync_copy(sums_v.at[pl.ds(0, ZB)], sums_sh.at[pl.ds(s * ZB, ZB)])

    agather.wait()
    pltpu.sync_copy(acorr_v, corr_sh.at[pl.ds(abase, AP)])

    jmin = jnp.where(is_last, OV16, 0)

    @plsc.parallel_loop(0, CH // L, unroll=8)
    def _vbody(j):
        id16 = id_v[pl.ds(j * L, L)]
        e16 = e_v[pl.ds(j * L, L)]
        keep = jnp.logical_and(id16 == c, j >= jmin)
        vals_v[pl.ds(j * L, L)] = jnp.where(keep, e16, jnp.zeros((L,), jnp.float32))

    plsc.subcore_barrier()

    pltpu.sync_copy(vals_v, sums_sh.at[sid_v], add=True)

    plsc.subcore_barrier()

    pltpu.sync_copy(sums_sh, sums_v)
    pltpu.sync_copy(corr_sh, corr_v)

    @plsc.parallel_loop(0, CH // L, unroll=8)
    def _gbody(j):
        sid16 = sid_v[pl.ds(j * L, L)]
        raw = plsc.load_gather(sums_v, [sid16])
        cfac = plsc.load_gather(corr_v, [sid16])
        raw_v[pl.ds(j * L, L)] = raw
        cor_v[pl.ds(j * L, L)] = raw * cfac

    @pl.when(c == 0)
    def _():
        pltpu.sync_copy(raw_v, hi_raw_h.at[pl.ds(base, CH)])
        pltpu.sync_copy(cor_v, hi_cor_h.at[pl.ds(base, CH)])

    @pl.when(c == 1)
    def _():
        pltpu.sync_copy(raw_v, tr_raw_h.at[pl.ds(base, CH)])
        pltpu.sync_copy(cor_v, tr_cor_h.at[pl.ds(base, CH)])


def kernel(pred_sid, pred_energy_corr_factor, pred_beta, recHitEnergy,
           recHitID, alpha_idx_tracks, alpha_idx_hits):
    del pred_beta
    alpha_all = jnp.concatenate([alpha_idx_hits.astype(jnp.int32),
                                 alpha_idx_tracks.astype(jnp.int32)])
    tr_raw, tr_cor, hi_raw, hi_cor = _sc_kernel(
        pred_sid.reshape(N), recHitID.reshape(N), recHitEnergy.reshape(N),
        pred_energy_corr_factor.reshape(N), alpha_all)
    out = lambda a: a.reshape(N, 1)
    return (out(tr_raw), out(tr_cor), out(hi_raw), out(hi_cor))
